# baseline (speedup 1.0000x reference)
import functools

import jax
import jax.numpy as jnp
from jax import lax
from jax.experimental import pallas as pl
from jax.experimental.pallas import tpu as pltpu
from jax.experimental.pallas import tpu_sc as plsc

B = 16384
L = 26
V = 1000000
D = 16

NC = 2
NS = 16
NW = NC * NS
NB = B // NW
CB = 64
CI = CB * L
GW = 128
NG = CI // GW
NCHUNK = NB // CB

NTILE = V // 128
TAILV = V - NTILE * 128


def _sc_detile_fn():
    mesh = plsc.VectorSubcoreMesh(core_axis_name="c", subcore_axis_name="s",
                                  num_cores=NC, num_subcores=NS)
    nloop = (NTILE + NW - 1) // NW

    @functools.partial(
        pl.kernel,
        out_type=jax.ShapeDtypeStruct((V * D,), jnp.float32),
        mesh=mesh,
        scratch_types=[
            pltpu.VMEM((8, 128), jnp.float32),
            pltpu.VMEM((8, 128), jnp.float32),
            pltpu.VMEM((8, 128), jnp.float32),
            pltpu.VMEM((8, 128), jnp.float32),
            pltpu.VMEM((512,), jnp.float32),
            pltpu.VMEM((512,), jnp.float32),
            pltpu.VMEM((512,), jnp.float32),
            pltpu.VMEM((512,), jnp.float32),
            pltpu.VMEM((512,), jnp.float32),
            pltpu.VMEM((512,), jnp.float32),
            pltpu.VMEM((512,), jnp.float32),
            pltpu.VMEM((512,), jnp.float32),
            pltpu.VMEM((8, TAILV), jnp.float32),
            pltpu.VMEM((8, TAILV), jnp.float32),
            pltpu.SemaphoreType.DMA,
            pltpu.SemaphoreType.DMA,
            pltpu.SemaphoreType.DMA,
            pltpu.SemaphoreType.DMA,
        ],
        compiler_params=pltpu.CompilerParams(
            needs_layout_passes=False, use_tc_tiling_on_sc=True),
    )
    def sc_detile(tab_hbm, out_hbm, bufa0_v, bufa1_v, bufb0_v, bufb1_v,
                  oq00, oq01, oq02, oq03, oq10, oq11, oq12, oq13,
                  ta_v, tb_v,
                  isem0, isem1, osem0, osem1):
        wid = lax.axis_index("s") * NC + lax.axis_index("c")
        lane = lax.iota(jnp.int32, 16)
        bufa = (bufa0_v, bufa1_v)
        bufb = (bufb0_v, bufb1_v)
        outq = ((oq00, oq01, oq02, oq03), (oq10, oq11, oq12, oq13))
        isems = (isem0, isem1)
        osems = (osem0, osem1)

        def start_in(c, par):
            l0 = pl.multiple_of(c * 128, 128)
            pltpu.async_copy(
                tab_hbm.at[0, :, pl.ds(l0, 128)], bufa[par], isems[par])
            pltpu.async_copy(
                tab_hbm.at[1, :, pl.ds(l0, 128)], bufb[par], isems[par])

        def wait_in(c, par):
            l0 = pl.multiple_of(c * 128, 128)
            pltpu.make_async_copy(
                tab_hbm.at[0, :, pl.ds(l0, 128)], bufa[par],
                isems[par]).wait()
            pltpu.make_async_copy(
                tab_hbm.at[1, :, pl.ds(l0, 128)], bufb[par],
                isems[par]).wait()

        def drain_out(par):
            for q in range(4):
                pltpu.make_async_copy(
                    outq[par][q], out_hbm.at[pl.ds(0, 512)],
                    osems[par]).wait()

        start_in(wid, 0)

        def body(jj, carry):
            for par in range(2):
                j = jj * 2 + par
                c = j * NW + wid
                cn = c + NW

                @pl.when(cn < NTILE)
                def _():
                    start_in(cn, 1 - par)

                @pl.when(c < NTILE)
                def _():
                    wait_in(c, par)

                    @pl.when(j >= 2)
                    def _():
                        drain_out(par)

                    lane16 = lane * D
                    for g in range(8):
                        q = outq[par][g // 2]
                        for d in range(8):
                            va = bufa[par][d, pl.ds(g * 16, 16)]
                            plsc.store_scatter(
                                q, [lane16 + ((g % 2) * 256) + d], va)
                            vb = bufb[par][d, pl.ds(g * 16, 16)]
                            plsc.store_scatter(
                                q, [lane16 + ((g % 2) * 256 + 8) + d], vb)

                    l0 = pl.multiple_of(c * 128, 128)
                    for q in range(4):
                        pltpu.async_copy(
                            outq[par][q],
                            out_hbm.at[pl.ds(l0 * D + q * 512, 512)],
                            osems[par])

            return carry

        lax.fori_loop(0, (nloop + 1) // 2, body, 0)
        drain_out(0)
        drain_out(1)

        @pl.when(wid == 0)
        def _():
            l0 = NTILE * 128
            cpa = pltpu.async_copy(
                tab_hbm.at[0, :, pl.ds(l0, TAILV)], ta_v, isem0)
            cpb = pltpu.async_copy(
                tab_hbm.at[1, :, pl.ds(l0, TAILV)], tb_v, isem0)
            cpa.wait()
            cpb.wait()
            lane16 = lane * D
            for g in range(TAILV // 16):
                q = outq[0][g // 2]
                for d in range(8):
                    va = ta_v[d, pl.ds(g * 16, 16)]
                    plsc.store_scatter(
                        q, [lane16 + ((g % 2) * 256) + d], va)
                    vb = tb_v[d, pl.ds(g * 16, 16)]
                    plsc.store_scatter(
                        q, [lane16 + ((g % 2) * 256 + 8) + d], vb)
            for q in range(TAILV * D // 512):
                pltpu.sync_copy(
                    outq[0][q],
                    out_hbm.at[pl.ds(l0 * D + q * 512, 512)])

    return sc_detile


_sc_detile = _sc_detile_fn()


def _sc_pool_fn():
    mesh = plsc.VectorSubcoreMesh(core_axis_name="c", subcore_axis_name="s",
                                  num_cores=NC, num_subcores=NS)

    @functools.partial(
        pl.kernel,
        out_type=(jax.ShapeDtypeStruct((B, D), jnp.float32),
                  jax.ShapeDtypeStruct((B, D), jnp.float32)),
        mesh=mesh,
        scratch_types=[
            pltpu.VMEM((CI,), jnp.int32),
            pltpu.VMEM((CI,), jnp.int32),
            pltpu.VMEM((CI,), jnp.float32),
            pltpu.VMEM((CI, D), jnp.float32),
            pltpu.VMEM((CI, D), jnp.float32),
            pltpu.VMEM((CB, D), jnp.float32),
            pltpu.VMEM((CB, D), jnp.float32),
            pltpu.SemaphoreType.DMA,
            pltpu.SemaphoreType.DMA,
        ],
        compiler_params=pltpu.CompilerParams(
            needs_layout_passes=False, use_tc_tiling_on_sc=False),
    )
    def sc_pool(x1_hbm, x2_hbm, x4_hbm, table_hbm, v1_hbm, v2_hbm,
                idx1_v, idx2_v, w_v, rows1_v, rows2_v, v1_v, v2_v,
                sem1, sem2):
        wid = lax.axis_index("s") * NC + lax.axis_index("c")

        def chunk_body(ci, carry):
            b0 = wid * NB + ci * CB
            i0 = b0 * L
            pltpu.sync_copy(x1_hbm.at[pl.ds(i0, CI)], idx1_v)
            pltpu.sync_copy(x2_hbm.at[pl.ds(i0, CI)], idx2_v)
            pltpu.sync_copy(x4_hbm.at[pl.ds(i0, CI)], w_v)
            cps = []
            for j in range(NG):
                cps.append(pltpu.async_copy(
                    table_hbm.at[idx1_v.at[pl.ds(j * GW, GW)]],
                    rows1_v.at[pl.ds(j * GW, GW)], sem1))
                cps.append(pltpu.async_copy(
                    table_hbm.at[idx2_v.at[pl.ds(j * GW, GW)]],
                    rows2_v.at[pl.ds(j * GW, GW)], sem2))
            for cp in cps:
                cp.wait()

            def b_body(b, carry2):
                r0 = b * L
                acc1 = jnp.zeros((D,), jnp.float32)
                acc2 = jnp.zeros((D,), jnp.float32)
                for l in range(L):
                    w = plsc.load_gather(
                        w_v, [jnp.full((D,), r0 + l, jnp.int32)])
                    acc1 = acc1 + rows1_v[r0 + l] * w
                    acc2 = acc2 + rows2_v[r0 + l]
                v1_v[b] = acc1
                v2_v[b] = acc2
                return carry2

            lax.fori_loop(0, CB, b_body, 0)
            pltpu.sync_copy(v1_v, v1_hbm.at[pl.ds(b0, CB)])
            pltpu.sync_copy(v2_v, v2_hbm.at[pl.ds(b0, CB)])
            return carry

        lax.fori_loop(0, NCHUNK, chunk_body, 0)

    return sc_pool


_sc_pool = _sc_pool_fn()


def _tc_finalize_body(v1_ref, v2_ref, o_ref):
    o_ref[...] = jax.nn.sigmoid(jnp.sum(v1_ref[...] * v2_ref[...], axis=1))


_tc_finalize = pl.pallas_call(
    _tc_finalize_body,
    out_shape=jax.ShapeDtypeStruct((B,), jnp.float32),
)


def kernel(x1, x2, x3, x4, table):
    del x3
    tab3 = table.T.reshape(2, 8, V)
    tab_lin = _sc_detile(tab3).reshape(V, D)
    x1f = x1.reshape(B * L)
    x2f = x2.reshape(B * L)
    x4f = x4.reshape(B * L)
    v1, v2 = _sc_pool(x1f, x2f, x4f, tab_lin)
    return _tc_finalize(v1, v2)


# speedup vs baseline: 1.0058x; 1.0058x over previous
import functools

import jax
import jax.numpy as jnp
from jax import lax
from jax.experimental import pallas as pl
from jax.experimental.pallas import tpu as pltpu
from jax.experimental.pallas import tpu_sc as plsc

B = 16384
L = 26
V = 1000000
D = 16

NC = 2
NS = 16
NW = NC * NS
NB = B // NW
CB = 64
CI = CB * L
GW = 128
NG = CI // GW
NCHUNK = NB // CB

NTILE = V // 128
TAILV = V - NTILE * 128


def _sc_detile_fn():
    mesh = plsc.VectorSubcoreMesh(core_axis_name="c", subcore_axis_name="s",
                                  num_cores=NC, num_subcores=NS)
    nloop = (NTILE + NW - 1) // NW

    @functools.partial(
        pl.kernel,
        out_type=jax.ShapeDtypeStruct((V * D,), jnp.float32),
        mesh=mesh,
        scratch_types=[
            pltpu.VMEM((8, 128), jnp.float32),
            pltpu.VMEM((8, 128), jnp.float32),
            pltpu.VMEM((8, 128), jnp.float32),
            pltpu.VMEM((8, 128), jnp.float32),
            pltpu.VMEM((2048,), jnp.float32),
            pltpu.VMEM((2048,), jnp.float32),
            pltpu.VMEM((8, TAILV), jnp.float32),
            pltpu.VMEM((8, TAILV), jnp.float32),
            pltpu.SemaphoreType.DMA,
            pltpu.SemaphoreType.DMA,
            pltpu.SemaphoreType.DMA,
            pltpu.SemaphoreType.DMA,
        ],
        compiler_params=pltpu.CompilerParams(
            needs_layout_passes=False, use_tc_tiling_on_sc=True),
    )
    def sc_detile(tab_hbm, out_hbm, bufa0_v, bufa1_v, bufb0_v, bufb1_v,
                  outb0_v, outb1_v, ta_v, tb_v,
                  isem0, isem1, osem0, osem1):
        wid = lax.axis_index("s") * NC + lax.axis_index("c")
        lane = lax.iota(jnp.int32, 16)
        bufa = (bufa0_v, bufa1_v)
        bufb = (bufb0_v, bufb1_v)
        outb = (outb0_v, outb1_v)
        isems = (isem0, isem1)
        osems = (osem0, osem1)

        def start_in(c, par):
            l0 = pl.multiple_of(c * 128, 128)
            pltpu.async_copy(
                tab_hbm.at[0, :, pl.ds(l0, 128)], bufa[par], isems[par])
            pltpu.async_copy(
                tab_hbm.at[1, :, pl.ds(l0, 128)], bufb[par], isems[par])

        def wait_in(c, par):
            l0 = pl.multiple_of(c * 128, 128)
            pltpu.make_async_copy(
                tab_hbm.at[0, :, pl.ds(l0, 128)], bufa[par],
                isems[par]).wait()
            pltpu.make_async_copy(
                tab_hbm.at[1, :, pl.ds(l0, 128)], bufb[par],
                isems[par]).wait()

        def drain_out(par):
            pltpu.make_async_copy(
                outb[par], out_hbm.at[pl.ds(0, 2048)],
                osems[par]).wait()

        start_in(wid, 0)

        def body(jj, carry):
            for par in range(2):
                j = jj * 2 + par
                c = j * NW + wid
                cn = c + NW

                @pl.when(cn < NTILE)
                def _():
                    start_in(cn, 1 - par)

                @pl.when(c < NTILE)
                def _():
                    wait_in(c, par)

                    @pl.when(j >= 2)
                    def _():
                        drain_out(par)

                    lane16 = lane * D

                    def gbody(g, carry2):
                        g16 = pl.multiple_of(g * 16, 16)
                        base = g * 256
                        for d in range(8):
                            va = bufa[par][d, pl.ds(g16, 16)]
                            plsc.store_scatter(
                                outb[par], [lane16 + base + d], va)
                            vb = bufb[par][d, pl.ds(g16, 16)]
                            plsc.store_scatter(
                                outb[par], [lane16 + base + 8 + d], vb)
                        return carry2

                    lax.fori_loop(0, 8, gbody, 0)
                    l0 = pl.multiple_of(c * 128, 128)
                    pltpu.async_copy(
                        outb[par], out_hbm.at[pl.ds(l0 * D, 2048)],
                        osems[par])

            return carry

        lax.fori_loop(0, (nloop + 1) // 2, body, 0)
        drain_out(0)
        drain_out(1)

        @pl.when(wid == 0)
        def _():
            l0 = NTILE * 128
            cpa = pltpu.async_copy(
                tab_hbm.at[0, :, pl.ds(l0, TAILV)], ta_v, isem0)
            cpb = pltpu.async_copy(
                tab_hbm.at[1, :, pl.ds(l0, TAILV)], tb_v, isem0)
            cpa.wait()
            cpb.wait()
            lane16 = lane * D
            for g in range(TAILV // 16):
                for d in range(8):
                    va = ta_v[d, pl.ds(g * 16, 16)]
                    plsc.store_scatter(
                        outb0_v, [lane16 + g * 256 + d], va)
                    vb = tb_v[d, pl.ds(g * 16, 16)]
                    plsc.store_scatter(
                        outb0_v, [lane16 + g * 256 + 8 + d], vb)
            pltpu.sync_copy(outb0_v.at[pl.ds(0, TAILV * D)],
                            out_hbm.at[pl.ds(l0 * D, TAILV * D)])

    return sc_detile


_sc_detile = _sc_detile_fn()


def _sc_pool_fn():
    mesh = plsc.VectorSubcoreMesh(core_axis_name="c", subcore_axis_name="s",
                                  num_cores=NC, num_subcores=NS)

    @functools.partial(
        pl.kernel,
        out_type=(jax.ShapeDtypeStruct((B, D), jnp.float32),
                  jax.ShapeDtypeStruct((B, D), jnp.float32)),
        mesh=mesh,
        scratch_types=[
            pltpu.VMEM((CI,), jnp.int32),
            pltpu.VMEM((CI,), jnp.int32),
            pltpu.VMEM((CI,), jnp.float32),
            pltpu.VMEM((CI, D), jnp.float32),
            pltpu.VMEM((CI, D), jnp.float32),
            pltpu.VMEM((CB, D), jnp.float32),
            pltpu.VMEM((CB, D), jnp.float32),
            pltpu.SemaphoreType.DMA,
            pltpu.SemaphoreType.DMA,
        ],
        compiler_params=pltpu.CompilerParams(
            needs_layout_passes=False, use_tc_tiling_on_sc=False),
    )
    def sc_pool(x1_hbm, x2_hbm, x4_hbm, table_hbm, v1_hbm, v2_hbm,
                idx1_v, idx2_v, w_v, rows1_v, rows2_v, v1_v, v2_v,
                sem1, sem2):
        wid = lax.axis_index("s") * NC + lax.axis_index("c")

        def chunk_body(ci, carry):
            b0 = wid * NB + ci * CB
            i0 = b0 * L
            pltpu.sync_copy(x1_hbm.at[pl.ds(i0, CI)], idx1_v)
            pltpu.sync_copy(x2_hbm.at[pl.ds(i0, CI)], idx2_v)
            pltpu.sync_copy(x4_hbm.at[pl.ds(i0, CI)], w_v)
            cps = []
            for j in range(NG):
                cps.append(pltpu.async_copy(
                    table_hbm.at[idx1_v.at[pl.ds(j * GW, GW)]],
                    rows1_v.at[pl.ds(j * GW, GW)], sem1))
                cps.append(pltpu.async_copy(
                    table_hbm.at[idx2_v.at[pl.ds(j * GW, GW)]],
                    rows2_v.at[pl.ds(j * GW, GW)], sem2))
            for cp in cps:
                cp.wait()

            def b_body(b, carry2):
                r0 = b * L
                acc1 = jnp.zeros((D,), jnp.float32)
                acc2 = jnp.zeros((D,), jnp.float32)
                for l in range(L):
                    w = plsc.load_gather(
                        w_v, [jnp.full((D,), r0 + l, jnp.int32)])
                    acc1 = acc1 + rows1_v[r0 + l] * w
                    acc2 = acc2 + rows2_v[r0 + l]
                v1_v[b] = acc1
                v2_v[b] = acc2
                return carry2

            lax.fori_loop(0, CB, b_body, 0)
            pltpu.sync_copy(v1_v, v1_hbm.at[pl.ds(b0, CB)])
            pltpu.sync_copy(v2_v, v2_hbm.at[pl.ds(b0, CB)])
            return carry

        lax.fori_loop(0, NCHUNK, chunk_body, 0)

    return sc_pool


_sc_pool = _sc_pool_fn()


def _tc_finalize_body(v1_ref, v2_ref, o_ref):
    o_ref[...] = jax.nn.sigmoid(jnp.sum(v1_ref[...] * v2_ref[...], axis=1))


_tc_finalize = pl.pallas_call(
    _tc_finalize_body,
    out_shape=jax.ShapeDtypeStruct((B,), jnp.float32),
)


def kernel(x1, x2, x3, x4, table):
    del x3
    tab3 = table.T.reshape(2, 8, V)
    tab_lin = _sc_detile(tab3).reshape(V, D)
    x1f = x1.reshape(B * L)
    x2f = x2.reshape(B * L)
    x4f = x4.reshape(B * L)
    v1, v2 = _sc_pool(x1f, x2f, x4f, tab_lin)
    return _tc_finalize(v1, v2)


# speedup vs baseline: 1.2045x; 1.1976x over previous
import functools

import jax
import jax.numpy as jnp
from jax import lax
from jax.experimental import pallas as pl
from jax.experimental.pallas import tpu as pltpu
from jax.experimental.pallas import tpu_sc as plsc

B = 16384
L = 26
V = 1000000
D = 16

NC = 2
NS = 16
NW = NC * NS
NB = B // NW
CB = 64
CI = CB * L
GW = 128
NG = CI // GW
NCHUNK = NB // CB

NTILE = V // 128
TAILV = V - NTILE * 128
SCW = 512
NSC = NTILE * 128 // SCW


def _sc_detile_fn():
    mesh = plsc.VectorSubcoreMesh(core_axis_name="c", subcore_axis_name="s",
                                  num_cores=NC, num_subcores=NS)
    nloop = (NSC + NW - 1) // NW

    @functools.partial(
        pl.kernel,
        out_type=jax.ShapeDtypeStruct((V * D,), jnp.float32),
        mesh=mesh,
        scratch_types=[
            pltpu.VMEM((8, SCW), jnp.float32),
            pltpu.VMEM((8, SCW), jnp.float32),
            pltpu.VMEM((8, SCW), jnp.float32),
            pltpu.VMEM((8, SCW), jnp.float32),
            pltpu.VMEM((SCW * D,), jnp.float32),
            pltpu.VMEM((SCW * D,), jnp.float32),
            pltpu.VMEM((8, TAILV), jnp.float32),
            pltpu.VMEM((8, TAILV), jnp.float32),
            pltpu.SemaphoreType.DMA,
            pltpu.SemaphoreType.DMA,
            pltpu.SemaphoreType.DMA,
            pltpu.SemaphoreType.DMA,
        ],
        compiler_params=pltpu.CompilerParams(
            needs_layout_passes=False, use_tc_tiling_on_sc=True),
    )
    def sc_detile(tab_hbm, out_hbm, bufa0_v, bufa1_v, bufb0_v, bufb1_v,
                  outb0_v, outb1_v, ta_v, tb_v,
                  isem0, isem1, osem0, osem1):
        wid = lax.axis_index("s") * NC + lax.axis_index("c")
        lane = lax.iota(jnp.int32, 16)
        bufa = (bufa0_v, bufa1_v)
        bufb = (bufb0_v, bufb1_v)
        outb = (outb0_v, outb1_v)
        isems = (isem0, isem1)
        osems = (osem0, osem1)

        def start_in(c, par):
            l0 = pl.multiple_of(c * SCW, SCW)
            pltpu.async_copy(
                tab_hbm.at[0, :, pl.ds(l0, SCW)], bufa[par], isems[par])
            pltpu.async_copy(
                tab_hbm.at[1, :, pl.ds(l0, SCW)], bufb[par], isems[par])

        def wait_in(c, par):
            l0 = pl.multiple_of(c * SCW, SCW)
            pltpu.make_async_copy(
                tab_hbm.at[0, :, pl.ds(l0, SCW)], bufa[par],
                isems[par]).wait()
            pltpu.make_async_copy(
                tab_hbm.at[1, :, pl.ds(l0, SCW)], bufb[par],
                isems[par]).wait()

        def drain_out(par):
            pltpu.make_async_copy(
                outb[par], out_hbm.at[pl.ds(0, SCW * D)],
                osems[par]).wait()

        start_in(wid, 0)

        def body(jj, carry):
            for par in range(2):
                j = jj * 2 + par
                c = j * NW + wid
                cn = c + NW

                @pl.when(cn < NSC)
                def _():
                    start_in(cn, 1 - par)

                @pl.when(c < NSC)
                def _():
                    wait_in(c, par)

                    @pl.when(j >= 2)
                    def _():
                        drain_out(par)

                    lane16 = lane * D

                    def gbody(g, carry2):
                        g16 = pl.multiple_of(g * 16, 16)
                        base = g * 256
                        for d in range(8):
                            va = bufa[par][d, pl.ds(g16, 16)]
                            plsc.store_scatter(
                                outb[par], [lane16 + base + d], va)
                            vb = bufb[par][d, pl.ds(g16, 16)]
                            plsc.store_scatter(
                                outb[par], [lane16 + base + 8 + d], vb)
                        return carry2

                    lax.fori_loop(0, SCW // 16, gbody, 0)
                    l0 = pl.multiple_of(c * SCW, SCW)
                    pltpu.async_copy(
                        outb[par], out_hbm.at[pl.ds(l0 * D, SCW * D)],
                        osems[par])

            return carry

        lax.fori_loop(0, (nloop + 1) // 2, body, 0)
        drain_out(0)
        drain_out(1)

        @pl.when(wid == 0)
        def _():
            l0 = NTILE * 128
            cpa = pltpu.async_copy(
                tab_hbm.at[0, :, pl.ds(l0, TAILV)], ta_v, isem0)
            cpb = pltpu.async_copy(
                tab_hbm.at[1, :, pl.ds(l0, TAILV)], tb_v, isem0)
            cpa.wait()
            cpb.wait()
            lane16 = lane * D
            for g in range(TAILV // 16):
                for d in range(8):
                    va = ta_v[d, pl.ds(g * 16, 16)]
                    plsc.store_scatter(
                        outb0_v, [lane16 + g * 256 + d], va)
                    vb = tb_v[d, pl.ds(g * 16, 16)]
                    plsc.store_scatter(
                        outb0_v, [lane16 + g * 256 + 8 + d], vb)
            pltpu.sync_copy(outb0_v.at[pl.ds(0, TAILV * D)],
                            out_hbm.at[pl.ds(l0 * D, TAILV * D)])

    return sc_detile


_sc_detile = _sc_detile_fn()


def _sc_pool_fn():
    mesh = plsc.VectorSubcoreMesh(core_axis_name="c", subcore_axis_name="s",
                                  num_cores=NC, num_subcores=NS)

    @functools.partial(
        pl.kernel,
        out_type=(jax.ShapeDtypeStruct((B, D), jnp.float32),
                  jax.ShapeDtypeStruct((B, D), jnp.float32)),
        mesh=mesh,
        scratch_types=[
            pltpu.VMEM((CI,), jnp.int32),
            pltpu.VMEM((CI,), jnp.int32),
            pltpu.VMEM((CI,), jnp.float32),
            pltpu.VMEM((CI, D), jnp.float32),
            pltpu.VMEM((CI, D), jnp.float32),
            pltpu.VMEM((CB, D), jnp.float32),
            pltpu.VMEM((CB, D), jnp.float32),
            pltpu.SemaphoreType.DMA,
            pltpu.SemaphoreType.DMA,
        ],
        compiler_params=pltpu.CompilerParams(
            needs_layout_passes=False, use_tc_tiling_on_sc=False),
    )
    def sc_pool(x1_hbm, x2_hbm, x4_hbm, table_hbm, v1_hbm, v2_hbm,
                idx1_v, idx2_v, w_v, rows1_v, rows2_v, v1_v, v2_v,
                sem1, sem2):
        wid = lax.axis_index("s") * NC + lax.axis_index("c")

        def chunk_body(ci, carry):
            b0 = wid * NB + ci * CB
            i0 = b0 * L
            pltpu.sync_copy(x1_hbm.at[pl.ds(i0, CI)], idx1_v)
            pltpu.sync_copy(x2_hbm.at[pl.ds(i0, CI)], idx2_v)
            pltpu.sync_copy(x4_hbm.at[pl.ds(i0, CI)], w_v)
            cps = []
            for j in range(NG):
                cps.append(pltpu.async_copy(
                    table_hbm.at[idx1_v.at[pl.ds(j * GW, GW)]],
                    rows1_v.at[pl.ds(j * GW, GW)], sem1))
                cps.append(pltpu.async_copy(
                    table_hbm.at[idx2_v.at[pl.ds(j * GW, GW)]],
                    rows2_v.at[pl.ds(j * GW, GW)], sem2))
            for cp in cps:
                cp.wait()

            def b_body(b, carry2):
                r0 = b * L
                acc1 = jnp.zeros((D,), jnp.float32)
                acc2 = jnp.zeros((D,), jnp.float32)
                for l in range(L):
                    w = plsc.load_gather(
                        w_v, [jnp.full((D,), r0 + l, jnp.int32)])
                    acc1 = acc1 + rows1_v[r0 + l] * w
                    acc2 = acc2 + rows2_v[r0 + l]
                v1_v[b] = acc1
                v2_v[b] = acc2
                return carry2

            lax.fori_loop(0, CB, b_body, 0)
            pltpu.sync_copy(v1_v, v1_hbm.at[pl.ds(b0, CB)])
            pltpu.sync_copy(v2_v, v2_hbm.at[pl.ds(b0, CB)])
            return carry

        lax.fori_loop(0, NCHUNK, chunk_body, 0)

    return sc_pool


_sc_pool = _sc_pool_fn()


def _tc_finalize_body(v1_ref, v2_ref, o_ref):
    o_ref[...] = jax.nn.sigmoid(jnp.sum(v1_ref[...] * v2_ref[...], axis=1))


_tc_finalize = pl.pallas_call(
    _tc_finalize_body,
    out_shape=jax.ShapeDtypeStruct((B,), jnp.float32),
)


def kernel(x1, x2, x3, x4, table):
    del x3
    tab3 = table.T.reshape(2, 8, V)
    tab_lin = _sc_detile(tab3).reshape(V, D)
    x1f = x1.reshape(B * L)
    x2f = x2.reshape(B * L)
    x4f = x4.reshape(B * L)
    v1, v2 = _sc_pool(x1f, x2f, x4f, tab_lin)
    return _tc_finalize(v1, v2)


# speedup vs baseline: 1.3643x; 1.1327x over previous
import functools

import jax
import jax.numpy as jnp
from jax import lax
from jax.experimental import pallas as pl
from jax.experimental.pallas import tpu as pltpu
from jax.experimental.pallas import tpu_sc as plsc

B = 16384
L = 26
V = 1000000
D = 16

NC = 2
NS = 16
NW = NC * NS
NB = B // NW
CB = 64
CI = CB * L
GW = 128
NG = CI // GW
NCHUNK = NB // CB

NTILE = V // 128
TAILV = V - NTILE * 128
SCW = 512
NSC = NTILE * 128 // SCW


def _sc_detile_fn():
    mesh = plsc.VectorSubcoreMesh(core_axis_name="c", subcore_axis_name="s",
                                  num_cores=NC, num_subcores=NS)
    nloop = (NSC + NW - 1) // NW

    @functools.partial(
        pl.kernel,
        out_type=jax.ShapeDtypeStruct((V * D,), jnp.float32),
        mesh=mesh,
        scratch_types=[
            pltpu.VMEM((8, SCW), jnp.float32),
            pltpu.VMEM((8, SCW), jnp.float32),
            pltpu.VMEM((8, SCW), jnp.float32),
            pltpu.VMEM((8, SCW), jnp.float32),
            pltpu.VMEM((SCW * D,), jnp.float32),
            pltpu.VMEM((SCW * D,), jnp.float32),
            pltpu.VMEM((8, TAILV), jnp.float32),
            pltpu.VMEM((8, TAILV), jnp.float32),
            pltpu.SemaphoreType.DMA,
            pltpu.SemaphoreType.DMA,
            pltpu.SemaphoreType.DMA,
            pltpu.SemaphoreType.DMA,
        ],
        compiler_params=pltpu.CompilerParams(
            needs_layout_passes=False, use_tc_tiling_on_sc=True),
    )
    def sc_detile(tab_hbm, out_hbm, bufa0_v, bufa1_v, bufb0_v, bufb1_v,
                  outb0_v, outb1_v, ta_v, tb_v,
                  isem0, isem1, osem0, osem1):
        wid = lax.axis_index("s") * NC + lax.axis_index("c")
        lane = lax.iota(jnp.int32, 16)
        bufa = (bufa0_v, bufa1_v)
        bufb = (bufb0_v, bufb1_v)
        outb = (outb0_v, outb1_v)
        isems = (isem0, isem1)
        osems = (osem0, osem1)

        def start_in(c, par):
            l0 = pl.multiple_of(c * SCW, SCW)
            pltpu.async_copy(
                tab_hbm.at[0, :, pl.ds(l0, SCW)], bufa[par], isems[par])
            pltpu.async_copy(
                tab_hbm.at[1, :, pl.ds(l0, SCW)], bufb[par], isems[par])

        def wait_in(c, par):
            l0 = pl.multiple_of(c * SCW, SCW)
            pltpu.make_async_copy(
                tab_hbm.at[0, :, pl.ds(l0, SCW)], bufa[par],
                isems[par]).wait()
            pltpu.make_async_copy(
                tab_hbm.at[1, :, pl.ds(l0, SCW)], bufb[par],
                isems[par]).wait()

        def drain_out(par):
            pltpu.make_async_copy(
                outb[par], out_hbm.at[pl.ds(0, SCW * D)],
                osems[par]).wait()

        start_in(wid, 0)

        def body(jj, carry):
            for par in range(2):
                j = jj * 2 + par
                c = j * NW + wid
                cn = c + NW

                @pl.when(cn < NSC)
                def _():
                    start_in(cn, 1 - par)

                @pl.when(c < NSC)
                def _():
                    wait_in(c, par)

                    @pl.when(j >= 2)
                    def _():
                        drain_out(par)

                    lane16 = lane * D

                    def gbody(g, carry2):
                        g16 = pl.multiple_of(g * 16, 16)
                        base = g * 256
                        for d in range(8):
                            va = bufa[par][d, pl.ds(g16, 16)]
                            plsc.store_scatter(
                                outb[par], [lane16 + base + d], va)
                            vb = bufb[par][d, pl.ds(g16, 16)]
                            plsc.store_scatter(
                                outb[par], [lane16 + base + 8 + d], vb)
                        return carry2

                    lax.fori_loop(0, SCW // 16, gbody, 0)
                    l0 = pl.multiple_of(c * SCW, SCW)
                    pltpu.async_copy(
                        outb[par], out_hbm.at[pl.ds(l0 * D, SCW * D)],
                        osems[par])

            return carry

        lax.fori_loop(0, (nloop + 1) // 2, body, 0)
        drain_out(0)
        drain_out(1)

        @pl.when(wid == 0)
        def _():
            l0 = NTILE * 128
            cpa = pltpu.async_copy(
                tab_hbm.at[0, :, pl.ds(l0, TAILV)], ta_v, isem0)
            cpb = pltpu.async_copy(
                tab_hbm.at[1, :, pl.ds(l0, TAILV)], tb_v, isem0)
            cpa.wait()
            cpb.wait()
            lane16 = lane * D
            for g in range(TAILV // 16):
                for d in range(8):
                    va = ta_v[d, pl.ds(g * 16, 16)]
                    plsc.store_scatter(
                        outb0_v, [lane16 + g * 256 + d], va)
                    vb = tb_v[d, pl.ds(g * 16, 16)]
                    plsc.store_scatter(
                        outb0_v, [lane16 + g * 256 + 8 + d], vb)
            pltpu.sync_copy(outb0_v.at[pl.ds(0, TAILV * D)],
                            out_hbm.at[pl.ds(l0 * D, TAILV * D)])

    return sc_detile


_sc_detile = _sc_detile_fn()


def _sc_pool_fn():
    mesh = plsc.VectorSubcoreMesh(core_axis_name="c", subcore_axis_name="s",
                                  num_cores=NC, num_subcores=NS)

    @functools.partial(
        pl.kernel,
        out_type=(jax.ShapeDtypeStruct((B, D), jnp.float32),
                  jax.ShapeDtypeStruct((B, D), jnp.float32)),
        mesh=mesh,
        scratch_types=[
            pltpu.VMEM((CI,), jnp.int32),
            pltpu.VMEM((CI,), jnp.int32),
            pltpu.VMEM((CI,), jnp.int32),
            pltpu.VMEM((CI,), jnp.int32),
            pltpu.VMEM((CI,), jnp.float32),
            pltpu.VMEM((CI,), jnp.float32),
            pltpu.VMEM((CI, D), jnp.float32),
            pltpu.VMEM((CI, D), jnp.float32),
            pltpu.VMEM((CI, D), jnp.float32),
            pltpu.VMEM((CI, D), jnp.float32),
            pltpu.VMEM((CB, D), jnp.float32),
            pltpu.VMEM((CB, D), jnp.float32),
            pltpu.VMEM((CB, D), jnp.float32),
            pltpu.VMEM((CB, D), jnp.float32),
            pltpu.SemaphoreType.DMA,
            pltpu.SemaphoreType.DMA,
            pltpu.SemaphoreType.DMA,
            pltpu.SemaphoreType.DMA,
            pltpu.SemaphoreType.DMA,
            pltpu.SemaphoreType.DMA,
        ],
        compiler_params=pltpu.CompilerParams(
            needs_layout_passes=False, use_tc_tiling_on_sc=False),
    )
    def sc_pool(x1_hbm, x2_hbm, x4_hbm, table_hbm, v1_hbm, v2_hbm,
                idx1a, idx1b, idx2a, idx2b, wa, wb,
                r1a, r1b, r2a, r2b, v1a, v1b, v2a, v2b,
                xsem0, xsem1, gsem0, gsem1, osem0, osem1):
        wid = lax.axis_index("s") * NC + lax.axis_index("c")
        idx1 = (idx1a, idx1b)
        idx2 = (idx2a, idx2b)
        wv = (wa, wb)
        rows1 = (r1a, r1b)
        rows2 = (r2a, r2b)
        v1v = (v1a, v1b)
        v2v = (v2a, v2b)
        xsems = (xsem0, xsem1)
        gsems = (gsem0, gsem1)
        osems = (osem0, osem1)

        def i0_of(ci):
            return pl.multiple_of((wid * NB + ci * CB) * L, CI)

        def start_idx(ci, par):
            i0 = i0_of(ci)
            pltpu.async_copy(x1_hbm.at[pl.ds(i0, CI)], idx1[par], xsems[par])
            pltpu.async_copy(x2_hbm.at[pl.ds(i0, CI)], idx2[par], xsems[par])
            pltpu.async_copy(x4_hbm.at[pl.ds(i0, CI)], wv[par], xsems[par])

        def wait_idx(ci, par):
            i0 = i0_of(ci)
            pltpu.make_async_copy(
                x1_hbm.at[pl.ds(i0, CI)], idx1[par], xsems[par]).wait()
            pltpu.make_async_copy(
                x2_hbm.at[pl.ds(i0, CI)], idx2[par], xsems[par]).wait()
            pltpu.make_async_copy(
                x4_hbm.at[pl.ds(i0, CI)], wv[par], xsems[par]).wait()

        def start_gathers(par):
            for j in range(NG):
                pltpu.async_copy(
                    table_hbm.at[idx1[par].at[pl.ds(j * GW, GW)]],
                    rows1[par].at[pl.ds(j * GW, GW)], gsems[par])
                pltpu.async_copy(
                    table_hbm.at[idx2[par].at[pl.ds(j * GW, GW)]],
                    rows2[par].at[pl.ds(j * GW, GW)], gsems[par])

        def wait_gathers(par):
            for j in range(NG):
                pltpu.make_async_copy(
                    table_hbm.at[idx1[par].at[pl.ds(j * GW, GW)]],
                    rows1[par].at[pl.ds(j * GW, GW)], gsems[par]).wait()
                pltpu.make_async_copy(
                    table_hbm.at[idx2[par].at[pl.ds(j * GW, GW)]],
                    rows2[par].at[pl.ds(j * GW, GW)], gsems[par]).wait()

        def drain_vout(ci, par):
            b0 = pl.multiple_of(wid * NB + ci * CB, CB)
            pltpu.make_async_copy(
                v1v[par], v1_hbm.at[pl.ds(b0, CB)], osems[par]).wait()
            pltpu.make_async_copy(
                v2v[par], v2_hbm.at[pl.ds(b0, CB)], osems[par]).wait()

        start_idx(0, 0)
        wait_idx(0, 0)
        start_gathers(0)
        start_idx(1, 1)

        def body(jj, carry):
            for par in range(2):
                ci = jj * 2 + par
                wait_gathers(par)

                @pl.when(ci + 1 < NCHUNK)
                def _():
                    wait_idx(ci + 1, 1 - par)
                    start_gathers(1 - par)

                @pl.when(ci >= 2)
                def _():
                    drain_vout(ci, par)

                def b_body(b, carry2):
                    r0 = b * L
                    acc1 = jnp.zeros((D,), jnp.float32)
                    acc2 = jnp.zeros((D,), jnp.float32)
                    for l in range(L):
                        w = plsc.load_gather(
                            wv[par], [jnp.full((D,), r0 + l, jnp.int32)])
                        acc1 = acc1 + rows1[par][r0 + l] * w
                        acc2 = acc2 + rows2[par][r0 + l]
                    v1v[par][b] = acc1
                    v2v[par][b] = acc2
                    return carry2

                lax.fori_loop(0, CB, b_body, 0)
                b0 = pl.multiple_of(wid * NB + ci * CB, CB)
                pltpu.async_copy(v1v[par], v1_hbm.at[pl.ds(b0, CB)],
                                 osems[par])
                pltpu.async_copy(v2v[par], v2_hbm.at[pl.ds(b0, CB)],
                                 osems[par])

                @pl.when(ci + 2 < NCHUNK)
                def _():
                    start_idx(ci + 2, par)

            return carry

        lax.fori_loop(0, NCHUNK // 2, body, 0)
        drain_vout(0, 0)
        drain_vout(0, 1)

    return sc_pool


_sc_pool = _sc_pool_fn()


def _tc_finalize_body(v1_ref, v2_ref, o_ref):
    o_ref[...] = jax.nn.sigmoid(jnp.sum(v1_ref[...] * v2_ref[...], axis=1))


_tc_finalize = pl.pallas_call(
    _tc_finalize_body,
    out_shape=jax.ShapeDtypeStruct((B,), jnp.float32),
)


def kernel(x1, x2, x3, x4, table):
    del x3
    tab3 = table.T.reshape(2, 8, V)
    tab_lin = _sc_detile(tab3).reshape(V, D)
    x1f = x1.reshape(B * L)
    x2f = x2.reshape(B * L)
    x4f = x4.reshape(B * L)
    v1, v2 = _sc_pool(x1f, x2f, x4f, tab_lin)
    return _tc_finalize(v1, v2)


# speedup vs baseline: 1.3645x; 1.0001x over previous
import functools

import jax
import jax.numpy as jnp
from jax import lax
from jax.experimental import pallas as pl
from jax.experimental.pallas import tpu as pltpu
from jax.experimental.pallas import tpu_sc as plsc

B = 16384
L = 26
V = 1000000
D = 16

NC = 2
NS = 16
NW = NC * NS
NB = B // NW
CB = 64
CI = CB * L
GW = 128
NG = CI // GW
NCHUNK = NB // CB

NTILE = V // 128
TAILV = V - NTILE * 128
SCW = 768
NSC = NTILE * 128 // SCW


def _sc_detile_fn():
    mesh = plsc.VectorSubcoreMesh(core_axis_name="c", subcore_axis_name="s",
                                  num_cores=NC, num_subcores=NS)
    nloop = (NSC + NW - 1) // NW

    @functools.partial(
        pl.kernel,
        out_type=jax.ShapeDtypeStruct((V * D,), jnp.float32),
        mesh=mesh,
        scratch_types=[
            pltpu.VMEM((8, SCW), jnp.float32),
            pltpu.VMEM((8, SCW), jnp.float32),
            pltpu.VMEM((8, SCW), jnp.float32),
            pltpu.VMEM((8, SCW), jnp.float32),
            pltpu.VMEM((SCW * D,), jnp.float32),
            pltpu.VMEM((SCW * D,), jnp.float32),
            pltpu.VMEM((8, TAILV), jnp.float32),
            pltpu.VMEM((8, TAILV), jnp.float32),
            pltpu.SemaphoreType.DMA,
            pltpu.SemaphoreType.DMA,
            pltpu.SemaphoreType.DMA,
            pltpu.SemaphoreType.DMA,
        ],
        compiler_params=pltpu.CompilerParams(
            needs_layout_passes=False, use_tc_tiling_on_sc=True),
    )
    def sc_detile(tab_hbm, out_hbm, bufa0_v, bufa1_v, bufb0_v, bufb1_v,
                  outb0_v, outb1_v, ta_v, tb_v,
                  isem0, isem1, osem0, osem1):
        wid = lax.axis_index("s") * NC + lax.axis_index("c")
        lane = lax.iota(jnp.int32, 16)
        bufa = (bufa0_v, bufa1_v)
        bufb = (bufb0_v, bufb1_v)
        outb = (outb0_v, outb1_v)
        isems = (isem0, isem1)
        osems = (osem0, osem1)

        def start_in(c, par):
            l0 = pl.multiple_of(c * SCW, SCW)
            pltpu.async_copy(
                tab_hbm.at[0, :, pl.ds(l0, SCW)], bufa[par], isems[par])
            pltpu.async_copy(
                tab_hbm.at[1, :, pl.ds(l0, SCW)], bufb[par], isems[par])

        def wait_in(c, par):
            l0 = pl.multiple_of(c * SCW, SCW)
            pltpu.make_async_copy(
                tab_hbm.at[0, :, pl.ds(l0, SCW)], bufa[par],
                isems[par]).wait()
            pltpu.make_async_copy(
                tab_hbm.at[1, :, pl.ds(l0, SCW)], bufb[par],
                isems[par]).wait()

        def drain_out(par):
            pltpu.make_async_copy(
                outb[par], out_hbm.at[pl.ds(0, SCW * D)],
                osems[par]).wait()

        start_in(wid, 0)

        def body(jj, carry):
            for par in range(2):
                j = jj * 2 + par
                c = j * NW + wid
                cn = c + NW

                @pl.when(cn < NSC)
                def _():
                    start_in(cn, 1 - par)

                @pl.when(c < NSC)
                def _():
                    wait_in(c, par)

                    @pl.when(j >= 2)
                    def _():
                        drain_out(par)

                    lane16 = lane * D

                    def gbody(g, carry2):
                        g16 = pl.multiple_of(g * 16, 16)
                        base = g * 256
                        for d in range(8):
                            va = bufa[par][d, pl.ds(g16, 16)]
                            plsc.store_scatter(
                                outb[par], [lane16 + base + d], va)
                            vb = bufb[par][d, pl.ds(g16, 16)]
                            plsc.store_scatter(
                                outb[par], [lane16 + base + 8 + d], vb)
                        return carry2

                    lax.fori_loop(0, SCW // 16, gbody, 0)
                    l0 = pl.multiple_of(c * SCW, SCW)
                    pltpu.async_copy(
                        outb[par], out_hbm.at[pl.ds(l0 * D, SCW * D)],
                        osems[par])

            return carry

        lax.fori_loop(0, (nloop + 1) // 2, body, 0)
        drain_out(0)
        drain_out(1)

        @pl.when(wid == 0)
        def _():
            l0 = NTILE * 128
            cpa = pltpu.async_copy(
                tab_hbm.at[0, :, pl.ds(l0, TAILV)], ta_v, isem0)
            cpb = pltpu.async_copy(
                tab_hbm.at[1, :, pl.ds(l0, TAILV)], tb_v, isem0)
            cpa.wait()
            cpb.wait()
            lane16 = lane * D
            for g in range(TAILV // 16):
                for d in range(8):
                    va = ta_v[d, pl.ds(g * 16, 16)]
                    plsc.store_scatter(
                        outb0_v, [lane16 + g * 256 + d], va)
                    vb = tb_v[d, pl.ds(g * 16, 16)]
                    plsc.store_scatter(
                        outb0_v, [lane16 + g * 256 + 8 + d], vb)
            pltpu.sync_copy(outb0_v.at[pl.ds(0, TAILV * D)],
                            out_hbm.at[pl.ds(l0 * D, TAILV * D)])

    return sc_detile


_sc_detile = _sc_detile_fn()


def _sc_pool_fn():
    mesh = plsc.VectorSubcoreMesh(core_axis_name="c", subcore_axis_name="s",
                                  num_cores=NC, num_subcores=NS)

    @functools.partial(
        pl.kernel,
        out_type=(jax.ShapeDtypeStruct((B, D), jnp.float32),
                  jax.ShapeDtypeStruct((B, D), jnp.float32)),
        mesh=mesh,
        scratch_types=[
            pltpu.VMEM((CI,), jnp.int32),
            pltpu.VMEM((CI,), jnp.int32),
            pltpu.VMEM((CI,), jnp.int32),
            pltpu.VMEM((CI,), jnp.int32),
            pltpu.VMEM((CI,), jnp.float32),
            pltpu.VMEM((CI,), jnp.float32),
            pltpu.VMEM((CI, D), jnp.float32),
            pltpu.VMEM((CI, D), jnp.float32),
            pltpu.VMEM((CI, D), jnp.float32),
            pltpu.VMEM((CI, D), jnp.float32),
            pltpu.VMEM((CB, D), jnp.float32),
            pltpu.VMEM((CB, D), jnp.float32),
            pltpu.VMEM((CB, D), jnp.float32),
            pltpu.VMEM((CB, D), jnp.float32),
            pltpu.SemaphoreType.DMA,
            pltpu.SemaphoreType.DMA,
            pltpu.SemaphoreType.DMA,
            pltpu.SemaphoreType.DMA,
            pltpu.SemaphoreType.DMA,
            pltpu.SemaphoreType.DMA,
        ],
        compiler_params=pltpu.CompilerParams(
            needs_layout_passes=False, use_tc_tiling_on_sc=False),
    )
    def sc_pool(x1_hbm, x2_hbm, x4_hbm, table_hbm, v1_hbm, v2_hbm,
                idx1a, idx1b, idx2a, idx2b, wa, wb,
                r1a, r1b, r2a, r2b, v1a, v1b, v2a, v2b,
                xsem0, xsem1, gsem0, gsem1, osem0, osem1):
        wid = lax.axis_index("s") * NC + lax.axis_index("c")
        idx1 = (idx1a, idx1b)
        idx2 = (idx2a, idx2b)
        wv = (wa, wb)
        rows1 = (r1a, r1b)
        rows2 = (r2a, r2b)
        v1v = (v1a, v1b)
        v2v = (v2a, v2b)
        xsems = (xsem0, xsem1)
        gsems = (gsem0, gsem1)
        osems = (osem0, osem1)

        def i0_of(ci):
            return pl.multiple_of((wid * NB + ci * CB) * L, CI)

        def start_idx(ci, par):
            i0 = i0_of(ci)
            pltpu.async_copy(x1_hbm.at[pl.ds(i0, CI)], idx1[par], xsems[par])
            pltpu.async_copy(x2_hbm.at[pl.ds(i0, CI)], idx2[par], xsems[par])
            pltpu.async_copy(x4_hbm.at[pl.ds(i0, CI)], wv[par], xsems[par])

        def wait_idx(ci, par):
            i0 = i0_of(ci)
            pltpu.make_async_copy(
                x1_hbm.at[pl.ds(i0, CI)], idx1[par], xsems[par]).wait()
            pltpu.make_async_copy(
                x2_hbm.at[pl.ds(i0, CI)], idx2[par], xsems[par]).wait()
            pltpu.make_async_copy(
                x4_hbm.at[pl.ds(i0, CI)], wv[par], xsems[par]).wait()

        def start_gathers(par):
            for j in range(NG):
                pltpu.async_copy(
                    table_hbm.at[idx1[par].at[pl.ds(j * GW, GW)]],
                    rows1[par].at[pl.ds(j * GW, GW)], gsems[par])
                pltpu.async_copy(
                    table_hbm.at[idx2[par].at[pl.ds(j * GW, GW)]],
                    rows2[par].at[pl.ds(j * GW, GW)], gsems[par])

        def wait_gathers(par):
            for j in range(NG):
                pltpu.make_async_copy(
                    table_hbm.at[idx1[par].at[pl.ds(j * GW, GW)]],
                    rows1[par].at[pl.ds(j * GW, GW)], gsems[par]).wait()
                pltpu.make_async_copy(
                    table_hbm.at[idx2[par].at[pl.ds(j * GW, GW)]],
                    rows2[par].at[pl.ds(j * GW, GW)], gsems[par]).wait()

        def drain_vout(ci, par):
            b0 = pl.multiple_of(wid * NB + ci * CB, CB)
            pltpu.make_async_copy(
                v1v[par], v1_hbm.at[pl.ds(b0, CB)], osems[par]).wait()
            pltpu.make_async_copy(
                v2v[par], v2_hbm.at[pl.ds(b0, CB)], osems[par]).wait()

        start_idx(0, 0)
        wait_idx(0, 0)
        start_gathers(0)
        start_idx(1, 1)

        def body(jj, carry):
            for par in range(2):
                ci = jj * 2 + par
                wait_gathers(par)

                @pl.when(ci + 1 < NCHUNK)
                def _():
                    wait_idx(ci + 1, 1 - par)
                    start_gathers(1 - par)

                @pl.when(ci >= 2)
                def _():
                    drain_vout(ci, par)

                def b_body(b, carry2):
                    r0 = b * L
                    acc1 = jnp.zeros((D,), jnp.float32)
                    acc2 = jnp.zeros((D,), jnp.float32)
                    for l in range(L):
                        w = plsc.load_gather(
                            wv[par], [jnp.full((D,), r0 + l, jnp.int32)])
                        acc1 = acc1 + rows1[par][r0 + l] * w
                        acc2 = acc2 + rows2[par][r0 + l]
                    v1v[par][b] = acc1
                    v2v[par][b] = acc2
                    return carry2

                lax.fori_loop(0, CB, b_body, 0)
                b0 = pl.multiple_of(wid * NB + ci * CB, CB)
                pltpu.async_copy(v1v[par], v1_hbm.at[pl.ds(b0, CB)],
                                 osems[par])
                pltpu.async_copy(v2v[par], v2_hbm.at[pl.ds(b0, CB)],
                                 osems[par])

                @pl.when(ci + 2 < NCHUNK)
                def _():
                    start_idx(ci + 2, par)

            return carry

        lax.fori_loop(0, NCHUNK // 2, body, 0)
        drain_vout(0, 0)
        drain_vout(0, 1)

    return sc_pool


_sc_pool = _sc_pool_fn()


def _tc_finalize_body(v1_ref, v2_ref, o_ref):
    o_ref[...] = jax.nn.sigmoid(jnp.sum(v1_ref[...] * v2_ref[...], axis=1))


_tc_finalize = pl.pallas_call(
    _tc_finalize_body,
    out_shape=jax.ShapeDtypeStruct((B,), jnp.float32),
)


def kernel(x1, x2, x3, x4, table):
    del x3
    tab3 = table.T.reshape(2, 8, V)
    tab_lin = _sc_detile(tab3).reshape(V, D)
    x1f = x1.reshape(B * L)
    x2f = x2.reshape(B * L)
    x4f = x4.reshape(B * L)
    v1, v2 = _sc_pool(x1f, x2f, x4f, tab_lin)
    return _tc_finalize(v1, v2)


# speedup vs baseline: 1.3762x; 1.0085x over previous
import functools

import jax
import jax.numpy as jnp
from jax import lax
from jax.experimental import pallas as pl
from jax.experimental.pallas import tpu as pltpu
from jax.experimental.pallas import tpu_sc as plsc

B = 16384
L = 26
V = 1000000
D = 16

NC = 2
NS = 16
NW = NC * NS
NB = B // NW
CB = 64
CI = CB * L
GW = 128
NG = CI // GW
NCHUNK = NB // CB

NTILE = V // 128
TAILV = V - NTILE * 128
SCW = 768
NSC = NTILE * 128 // SCW


def _sc_detile_fn():
    mesh = plsc.VectorSubcoreMesh(core_axis_name="c", subcore_axis_name="s",
                                  num_cores=NC, num_subcores=NS)
    nloop = (NSC + NW - 1) // NW

    @functools.partial(
        pl.kernel,
        out_type=jax.ShapeDtypeStruct((V * D,), jnp.float32),
        mesh=mesh,
        scratch_types=[
            pltpu.VMEM((8, SCW), jnp.float32),
            pltpu.VMEM((8, SCW), jnp.float32),
            pltpu.VMEM((8, SCW), jnp.float32),
            pltpu.VMEM((8, SCW), jnp.float32),
            pltpu.VMEM((SCW * D,), jnp.float32),
            pltpu.VMEM((SCW * D,), jnp.float32),
            pltpu.VMEM((8, TAILV), jnp.float32),
            pltpu.VMEM((8, TAILV), jnp.float32),
            pltpu.SemaphoreType.DMA,
            pltpu.SemaphoreType.DMA,
            pltpu.SemaphoreType.DMA,
            pltpu.SemaphoreType.DMA,
        ],
        compiler_params=pltpu.CompilerParams(
            needs_layout_passes=False, use_tc_tiling_on_sc=True),
    )
    def sc_detile(tab_hbm, out_hbm, bufa0_v, bufa1_v, bufb0_v, bufb1_v,
                  outb0_v, outb1_v, ta_v, tb_v,
                  isem0, isem1, osem0, osem1):
        wid = lax.axis_index("s") * NC + lax.axis_index("c")
        lane = lax.iota(jnp.int32, 16)
        bufa = (bufa0_v, bufa1_v)
        bufb = (bufb0_v, bufb1_v)
        outb = (outb0_v, outb1_v)
        isems = (isem0, isem1)
        osems = (osem0, osem1)

        def start_in(c, par):
            l0 = pl.multiple_of(c * SCW, SCW)
            pltpu.async_copy(
                tab_hbm.at[0, :, pl.ds(l0, SCW)], bufa[par], isems[par])
            pltpu.async_copy(
                tab_hbm.at[1, :, pl.ds(l0, SCW)], bufb[par], isems[par])

        def wait_in(c, par):
            l0 = pl.multiple_of(c * SCW, SCW)
            pltpu.make_async_copy(
                tab_hbm.at[0, :, pl.ds(l0, SCW)], bufa[par],
                isems[par]).wait()
            pltpu.make_async_copy(
                tab_hbm.at[1, :, pl.ds(l0, SCW)], bufb[par],
                isems[par]).wait()

        def drain_out(par):
            pltpu.make_async_copy(
                outb[par], out_hbm.at[pl.ds(0, SCW * D)],
                osems[par]).wait()

        start_in(wid, 0)

        def body(jj, carry):
            for par in range(2):
                j = jj * 2 + par
                c = j * NW + wid
                cn = c + NW

                @pl.when(cn < NSC)
                def _():
                    start_in(cn, 1 - par)

                @pl.when(c < NSC)
                def _():
                    wait_in(c, par)

                    @pl.when(j >= 2)
                    def _():
                        drain_out(par)

                    lane16 = lane * D

                    def gbody(g, carry2):
                        g16 = pl.multiple_of(g * 16, 16)
                        base = g * 256
                        for d in range(8):
                            va = bufa[par][d, pl.ds(g16, 16)]
                            plsc.store_scatter(
                                outb[par], [lane16 + base + d], va)
                            vb = bufb[par][d, pl.ds(g16, 16)]
                            plsc.store_scatter(
                                outb[par], [lane16 + base + 8 + d], vb)
                        return carry2

                    lax.fori_loop(0, SCW // 16, gbody, 0)
                    l0 = pl.multiple_of(c * SCW, SCW)
                    pltpu.async_copy(
                        outb[par], out_hbm.at[pl.ds(l0 * D, SCW * D)],
                        osems[par])

            return carry

        lax.fori_loop(0, (nloop + 1) // 2, body, 0)
        drain_out(0)
        drain_out(1)

        @pl.when(wid == 0)
        def _():
            l0 = NTILE * 128
            cpa = pltpu.async_copy(
                tab_hbm.at[0, :, pl.ds(l0, TAILV)], ta_v, isem0)
            cpb = pltpu.async_copy(
                tab_hbm.at[1, :, pl.ds(l0, TAILV)], tb_v, isem0)
            cpa.wait()
            cpb.wait()
            lane16 = lane * D
            for g in range(TAILV // 16):
                for d in range(8):
                    va = ta_v[d, pl.ds(g * 16, 16)]
                    plsc.store_scatter(
                        outb0_v, [lane16 + g * 256 + d], va)
                    vb = tb_v[d, pl.ds(g * 16, 16)]
                    plsc.store_scatter(
                        outb0_v, [lane16 + g * 256 + 8 + d], vb)
            pltpu.sync_copy(outb0_v.at[pl.ds(0, TAILV * D)],
                            out_hbm.at[pl.ds(l0 * D, TAILV * D)])

    return sc_detile


_sc_detile = _sc_detile_fn()


def _sc_pool_fn():
    mesh = plsc.VectorSubcoreMesh(core_axis_name="c", subcore_axis_name="s",
                                  num_cores=NC, num_subcores=NS)

    @functools.partial(
        pl.kernel,
        out_type=(jax.ShapeDtypeStruct((B, D), jnp.float32),
                  jax.ShapeDtypeStruct((B, D), jnp.float32)),
        mesh=mesh,
        scratch_types=[
            pltpu.VMEM((CI,), jnp.int32),
            pltpu.VMEM((CI,), jnp.int32),
            pltpu.VMEM((CI,), jnp.int32),
            pltpu.VMEM((CI,), jnp.int32),
            pltpu.VMEM((CI,), jnp.float32),
            pltpu.VMEM((CI,), jnp.float32),
            pltpu.VMEM((CI, D), jnp.float32),
            pltpu.VMEM((CI, D), jnp.float32),
            pltpu.VMEM((CI, D), jnp.float32),
            pltpu.VMEM((CI, D), jnp.float32),
            pltpu.VMEM((CB, D), jnp.float32),
            pltpu.VMEM((CB, D), jnp.float32),
            pltpu.VMEM((CB, D), jnp.float32),
            pltpu.VMEM((CB, D), jnp.float32),
            pltpu.SemaphoreType.DMA,
            pltpu.SemaphoreType.DMA,
            pltpu.SemaphoreType.DMA,
            pltpu.SemaphoreType.DMA,
            pltpu.SemaphoreType.DMA,
            pltpu.SemaphoreType.DMA,
        ],
        compiler_params=pltpu.CompilerParams(
            needs_layout_passes=False, use_tc_tiling_on_sc=False),
    )
    def sc_pool(x1_hbm, x2_hbm, x4_hbm, table_hbm, v1_hbm, v2_hbm,
                idx1a, idx1b, idx2a, idx2b, wa, wb,
                r1a, r1b, r2a, r2b, v1a, v1b, v2a, v2b,
                xsem0, xsem1, gsem0, gsem1, osem0, osem1):
        wid = lax.axis_index("s") * NC + lax.axis_index("c")
        idx1 = (idx1a, idx1b)
        idx2 = (idx2a, idx2b)
        wv = (wa, wb)
        rows1 = (r1a, r1b)
        rows2 = (r2a, r2b)
        v1v = (v1a, v1b)
        v2v = (v2a, v2b)
        xsems = (xsem0, xsem1)
        gsems = (gsem0, gsem1)
        osems = (osem0, osem1)

        def i0_of(ci):
            return pl.multiple_of((wid * NB + ci * CB) * L, CI)

        def start_idx(ci, par):
            i0 = i0_of(ci)
            pltpu.async_copy(x1_hbm.at[pl.ds(i0, CI)], idx1[par], xsems[par])
            pltpu.async_copy(x2_hbm.at[pl.ds(i0, CI)], idx2[par], xsems[par])
            pltpu.async_copy(x4_hbm.at[pl.ds(i0, CI)], wv[par], xsems[par])

        def wait_idx(ci, par):
            i0 = i0_of(ci)
            pltpu.make_async_copy(
                x1_hbm.at[pl.ds(i0, CI)], idx1[par], xsems[par]).wait()
            pltpu.make_async_copy(
                x2_hbm.at[pl.ds(i0, CI)], idx2[par], xsems[par]).wait()
            pltpu.make_async_copy(
                x4_hbm.at[pl.ds(i0, CI)], wv[par], xsems[par]).wait()

        def start_gathers(par):
            for j in range(NG):
                pltpu.async_copy(
                    table_hbm.at[idx1[par].at[pl.ds(j * GW, GW)]],
                    rows1[par].at[pl.ds(j * GW, GW)], gsems[par])
                pltpu.async_copy(
                    table_hbm.at[idx2[par].at[pl.ds(j * GW, GW)]],
                    rows2[par].at[pl.ds(j * GW, GW)], gsems[par])

        def wait_gathers(par):
            for j in range(NG):
                pltpu.make_async_copy(
                    table_hbm.at[idx1[par].at[pl.ds(j * GW, GW)]],
                    rows1[par].at[pl.ds(j * GW, GW)], gsems[par]).wait()
                pltpu.make_async_copy(
                    table_hbm.at[idx2[par].at[pl.ds(j * GW, GW)]],
                    rows2[par].at[pl.ds(j * GW, GW)], gsems[par]).wait()

        def drain_vout(ci, par):
            b0 = pl.multiple_of(wid * NB + ci * CB, CB)
            pltpu.make_async_copy(
                v1v[par], v1_hbm.at[pl.ds(b0, CB)], osems[par]).wait()
            pltpu.make_async_copy(
                v2v[par], v2_hbm.at[pl.ds(b0, CB)], osems[par]).wait()

        start_idx(0, 0)
        wait_idx(0, 0)
        start_gathers(0)
        start_idx(1, 1)

        def body(jj, carry):
            for par in range(2):
                ci = jj * 2 + par
                wait_gathers(par)

                @pl.when(ci + 1 < NCHUNK)
                def _():
                    wait_idx(ci + 1, 1 - par)
                    start_gathers(1 - par)

                @pl.when(ci >= 2)
                def _():
                    drain_vout(ci, par)

                def b_body(b, carry2):
                    r0 = b * L
                    wv0 = wv[par][pl.ds(r0, 16)]
                    wv1 = wv[par][pl.ds(r0 + 16, 16)]
                    acc1 = jnp.zeros((D,), jnp.float32)
                    acc2 = jnp.zeros((D,), jnp.float32)
                    for l in range(L):
                        wl = wv0[l] if l < 16 else wv1[l - 16]
                        acc1 = acc1 + rows1[par][r0 + l] * wl
                        acc2 = acc2 + rows2[par][r0 + l]
                    v1v[par][b] = acc1
                    v2v[par][b] = acc2
                    return carry2

                lax.fori_loop(0, CB, b_body, 0)
                b0 = pl.multiple_of(wid * NB + ci * CB, CB)
                pltpu.async_copy(v1v[par], v1_hbm.at[pl.ds(b0, CB)],
                                 osems[par])
                pltpu.async_copy(v2v[par], v2_hbm.at[pl.ds(b0, CB)],
                                 osems[par])

                @pl.when(ci + 2 < NCHUNK)
                def _():
                    start_idx(ci + 2, par)

            return carry

        lax.fori_loop(0, NCHUNK // 2, body, 0)
        drain_vout(0, 0)
        drain_vout(0, 1)

    return sc_pool


_sc_pool = _sc_pool_fn()


def _tc_finalize_body(v1_ref, v2_ref, o_ref):
    o_ref[...] = jax.nn.sigmoid(jnp.sum(v1_ref[...] * v2_ref[...], axis=1))


_tc_finalize = pl.pallas_call(
    _tc_finalize_body,
    out_shape=jax.ShapeDtypeStruct((B,), jnp.float32),
)


def kernel(x1, x2, x3, x4, table):
    del x3
    tab3 = table.T.reshape(2, 8, V)
    tab_lin = _sc_detile(tab3).reshape(V, D)
    x1f = x1.reshape(B * L)
    x2f = x2.reshape(B * L)
    x4f = x4.reshape(B * L)
    v1, v2 = _sc_pool(x1f, x2f, x4f, tab_lin)
    return _tc_finalize(v1, v2)
